# Initial kernel scaffold; baseline (speedup 1.0000x reference)
import jax
import jax.numpy as jnp
from jax.experimental import pallas as pl


def kernel(node_pos, raw_edge_index, batch):
    raise NotImplementedError("write your pallas kernel here")


# speedup vs baseline: 39.3030x; 39.3030x over previous
import functools

import jax
import jax.numpy as jnp
from jax import lax
from jax.experimental import pallas as pl
from jax.experimental.pallas import tpu as pltpu
from jax.experimental.pallas import tpu_sc as plsc

N_NODES = 100000
N_EDGES = 6400000
N_GRAPHS = 128

ROW_W = 128
N_ROWS = N_EDGES // ROW_W
NW = 32
ACC = N_GRAPHS * 16


def _edge_var_partials(table, src2d, dst2d):
    mesh = plsc.VectorSubcoreMesh(core_axis_name="c", subcore_axis_name="s")

    @functools.partial(
        pl.kernel,
        mesh=mesh,
        compiler_params=pltpu.CompilerParams(
            needs_layout_passes=False, use_tc_tiling_on_sc=False
        ),
        out_type=(
            jax.ShapeDtypeStruct((NW, ACC), jnp.float32),
            jax.ShapeDtypeStruct((NW, ACC), jnp.float32),
        ),
        scratch_types=[
            pltpu.VMEM((ROW_W,), jnp.int32),
            pltpu.VMEM((ROW_W,), jnp.int32),
            pltpu.VMEM((ROW_W, 8), jnp.float32),
            pltpu.VMEM((ROW_W, 8), jnp.float32),
            pltpu.VMEM((ACC,), jnp.float32),
            pltpu.VMEM((ACC,), jnp.float32),
            pltpu.SemaphoreType.DMA,
            pltpu.SemaphoreType.DMA,
        ],
    )
    def k(table_h, src_h, dst_h, sums_o, cnts_o,
          idx_s, idx_d, rows_s, rows_d, acc_s, acc_c, sem0, sem1):
        cid = lax.axis_index("c")
        sid = lax.axis_index("s")
        wid = sid * 2 + cid

        lanes = lax.iota(jnp.int32, 16)
        zeros16 = jnp.zeros((16,), jnp.float32)
        ones16 = jnp.ones((16,), jnp.float32)

        def zero_body(i, _):
            acc_s[pl.ds(i * 16, 16)] = zeros16
            acc_c[pl.ds(i * 16, 16)] = zeros16
            return 0

        lax.fori_loop(0, ACC // 16, zero_body, 0)

        base = N_ROWS // NW
        extra = N_ROWS - base * NW
        start = wid * base + jnp.minimum(wid, extra)
        count = base + jnp.where(wid < extra, 1, 0)

        magic = jnp.full((16,), 0x5F3759DF, jnp.int32)
        half = jnp.full((16,), 0.5, jnp.float32)
        threehalf = jnp.full((16,), 1.5, jnp.float32)

        def row_body(g, _):
            r = start + g
            pltpu.sync_copy(src_h.at[r], idx_s)
            pltpu.sync_copy(dst_h.at[r], idx_d)
            c0 = pltpu.async_copy(table_h.at[idx_s], rows_s, sem0)
            c1 = pltpu.async_copy(table_h.at[idx_d], rows_d, sem1)
            c0.wait()
            c1.wait()
            col0 = jnp.full((16,), 0, jnp.int32)
            col1 = jnp.full((16,), 1, jnp.int32)
            col2 = jnp.full((16,), 2, jnp.int32)
            col3 = jnp.full((16,), 3, jnp.int32)
            for j in range(ROW_W // 16):
                ridx = lanes + (j * 16)
                sx = plsc.load_gather(rows_s, [ridx, col0])
                sy = plsc.load_gather(rows_s, [ridx, col1])
                sz = plsc.load_gather(rows_s, [ridx, col2])
                sb = plsc.load_gather(rows_s, [ridx, col3])
                dx = plsc.load_gather(rows_d, [ridx, col0])
                dy = plsc.load_gather(rows_d, [ridx, col1])
                dz = plsc.load_gather(rows_d, [ridx, col2])
                ex = dx - sx
                ey = dy - sy
                ez = dz - sz
                d2 = ex * ex + ey * ey + ez * ez
                yi = magic - lax.shift_right_arithmetic(plsc.bitcast(d2, jnp.int32),
                                                        jnp.full((16,), 1, jnp.int32))
                y = plsc.bitcast(yi, jnp.float32)
                hx = half * d2
                y = y * (threehalf - hx * y * y)
                y = y * (threehalf - hx * y * y)
                y = y * (threehalf - hx * y * y)
                eu = d2 * y
                em1 = eu - ones16
                ev = em1 * em1
                b = sb.astype(jnp.int32)
                slot = b * 16 + lanes
                plsc.addupdate_scatter(acc_s, [slot], ev)
                plsc.addupdate_scatter(acc_c, [slot], ones16)
            return 0

        lax.fori_loop(0, count, row_body, 0)

        pltpu.sync_copy(acc_s, sums_o.at[wid])
        pltpu.sync_copy(acc_c, cnts_o.at[wid])

    return k(table, src2d, dst2d)


def kernel(node_pos, raw_edge_index, batch):
    batch_i = batch.astype(jnp.int32)
    edges = raw_edge_index.astype(jnp.int32)
    table = jnp.concatenate(
        [
            node_pos.astype(jnp.float32),
            batch_i.astype(jnp.float32)[:, None],
            jnp.zeros((N_NODES, 4), jnp.float32),
        ],
        axis=1,
    )
    src2d = edges[0].reshape(N_ROWS, ROW_W)
    dst2d = edges[1].reshape(N_ROWS, ROW_W)
    sums_p, cnts_p = _edge_var_partials(table, src2d, dst2d)
    s = sums_p.sum(axis=0).reshape(N_GRAPHS, 16).sum(axis=1)
    c = cnts_p.sum(axis=0).reshape(N_GRAPHS, 16).sum(axis=1)
    graph_var = jnp.where(c > 0, s / jnp.maximum(c, 1.0), 0.0)
    return jnp.mean(graph_var)


# speedup vs baseline: 105.2888x; 2.6789x over previous
import functools

import jax
import jax.numpy as jnp
from jax import lax
from jax.experimental import pallas as pl
from jax.experimental.pallas import tpu as pltpu
from jax.experimental.pallas import tpu_sc as plsc

N_NODES = 100000
N_EDGES = 6400000
N_GRAPHS = 128

ROW_W = 128
NW = 32
CH = 16
R_PAD = ((N_EDGES // ROW_W) + NW * CH - 1) // (NW * CH) * (NW * CH)
ROWS_PER_W = R_PAD // NW
NCH = ROWS_PER_W // CH
ACC = (N_GRAPHS + 1) * 16


def _edge_var_partials(table, src2d, dst2d):
    mesh = plsc.VectorSubcoreMesh(core_axis_name="c", subcore_axis_name="s")

    @functools.partial(
        pl.kernel,
        mesh=mesh,
        compiler_params=pltpu.CompilerParams(
            needs_layout_passes=False, use_tc_tiling_on_sc=False
        ),
        out_type=(
            jax.ShapeDtypeStruct((NW, ACC), jnp.float32),
            jax.ShapeDtypeStruct((NW, ACC), jnp.float32),
        ),
        scratch_types=[
            pltpu.VMEM((2, CH, ROW_W), jnp.int32),
            pltpu.VMEM((2, CH, ROW_W), jnp.int32),
            pltpu.VMEM((2, CH * ROW_W, 8), jnp.float32),
            pltpu.VMEM((2, CH * ROW_W, 8), jnp.float32),
            pltpu.VMEM((ACC,), jnp.float32),
            pltpu.VMEM((ACC,), jnp.float32),
            pltpu.SemaphoreType.DMA,
            pltpu.SemaphoreType.DMA,
        ],
    )
    def k(table_h, src_h, dst_h, sums_o, cnts_o,
          idx_s, idx_d, rows_s, rows_d, acc_s, acc_c, sem_s, sem_d):
        cid = lax.axis_index("c")
        sid = lax.axis_index("s")
        wid = sid * 2 + cid

        lanes = lax.iota(jnp.int32, 16)
        zeros16 = jnp.zeros((16,), jnp.float32)
        ones16 = jnp.ones((16,), jnp.float32)

        def zero_body(i, _):
            acc_s[pl.ds(i * 16, 16)] = zeros16
            acc_c[pl.ds(i * 16, 16)] = zeros16
            return 0

        lax.fori_loop(0, ACC // 16, zero_body, 0)

        start = wid * ROWS_PER_W

        def fire(t, buf):
            row = start + t * CH
            pltpu.sync_copy(src_h.at[pl.ds(row, CH)], idx_s.at[buf])
            pltpu.sync_copy(dst_h.at[pl.ds(row, CH)], idx_d.at[buf])
            for j in range(CH):
                pltpu.async_copy(table_h.at[idx_s.at[buf, j]],
                                 rows_s.at[buf, pl.ds(j * ROW_W, ROW_W)], sem_s)
                pltpu.async_copy(table_h.at[idx_d.at[buf, j]],
                                 rows_d.at[buf, pl.ds(j * ROW_W, ROW_W)], sem_d)

        def drain(buf):
            for j in range(CH):
                pltpu.make_async_copy(
                    table_h.at[idx_s.at[buf, j]],
                    rows_s.at[buf, pl.ds(j * ROW_W, ROW_W)], sem_s).wait()
                pltpu.make_async_copy(
                    table_h.at[idx_d.at[buf, j]],
                    rows_d.at[buf, pl.ds(j * ROW_W, ROW_W)], sem_d).wait()

        magic = jnp.full((16,), 0x5F3759DF, jnp.int32)
        one_i = jnp.full((16,), 1, jnp.int32)
        half = jnp.full((16,), 0.5, jnp.float32)
        threehalf = jnp.full((16,), 1.5, jnp.float32)
        col0 = jnp.full((16,), 0, jnp.int32)
        col1 = jnp.full((16,), 1, jnp.int32)
        col2 = jnp.full((16,), 2, jnp.int32)
        col3 = jnp.full((16,), 3, jnp.int32)

        fire(0, 0)

        def chunk_body(t, _):
            buf = lax.rem(t, 2)
            drain(buf)

            @pl.when(t + 1 < NCH)
            def _():
                fire(t + 1, 1 - buf)

            srows = rows_s.at[buf]
            drows = rows_d.at[buf]
            for j in range(CH * ROW_W // 16):
                ridx = lanes + (j * 16)
                sx = plsc.load_gather(srows, [ridx, col0])
                sy = plsc.load_gather(srows, [ridx, col1])
                sz = plsc.load_gather(srows, [ridx, col2])
                sb = plsc.load_gather(srows, [ridx, col3])
                dx = plsc.load_gather(drows, [ridx, col0])
                dy = plsc.load_gather(drows, [ridx, col1])
                dz = plsc.load_gather(drows, [ridx, col2])
                ex = dx - sx
                ey = dy - sy
                ez = dz - sz
                d2 = ex * ex + ey * ey + ez * ez
                yi = magic - lax.shift_right_arithmetic(
                    plsc.bitcast(d2, jnp.int32), one_i)
                y = plsc.bitcast(yi, jnp.float32)
                hx = half * d2
                y = y * (threehalf - hx * y * y)
                y = y * (threehalf - hx * y * y)
                y = y * (threehalf - hx * y * y)
                eu = d2 * y
                em1 = eu - ones16
                ev = em1 * em1
                b = sb.astype(jnp.int32)
                slot = b * 16 + lanes
                plsc.addupdate_scatter(acc_s, [slot], ev)
                plsc.addupdate_scatter(acc_c, [slot], ones16)
            return 0

        lax.fori_loop(0, NCH, chunk_body, 0)

        pltpu.sync_copy(acc_s, sums_o.at[wid])
        pltpu.sync_copy(acc_c, cnts_o.at[wid])

    return k(table, src2d, dst2d)


def kernel(node_pos, raw_edge_index, batch):
    batch_i = batch.astype(jnp.int32)
    edges = raw_edge_index.astype(jnp.int32)
    sentinel = jnp.array([[0.0, 0.0, 0.0, float(N_GRAPHS)]], jnp.float32)
    table = jnp.concatenate(
        [
            jnp.concatenate(
                [node_pos.astype(jnp.float32),
                 batch_i.astype(jnp.float32)[:, None]], axis=1
            ),
            sentinel,
        ],
        axis=0,
    )
    table = jnp.concatenate(
        [table, jnp.zeros((N_NODES + 1, 4), jnp.float32)], axis=1
    )
    n_pad = R_PAD * ROW_W - N_EDGES
    pad = jnp.full((n_pad,), N_NODES, jnp.int32)
    src2d = jnp.concatenate([edges[0], pad]).reshape(R_PAD, ROW_W)
    dst2d = jnp.concatenate([edges[1], pad]).reshape(R_PAD, ROW_W)
    sums_p, cnts_p = _edge_var_partials(table, src2d, dst2d)
    s = sums_p.sum(axis=0)[: N_GRAPHS * 16].reshape(N_GRAPHS, 16).sum(axis=1)
    c = cnts_p.sum(axis=0)[: N_GRAPHS * 16].reshape(N_GRAPHS, 16).sum(axis=1)
    graph_var = jnp.where(c > 0, s / jnp.maximum(c, 1.0), 0.0)
    return jnp.mean(graph_var)


# speedup vs baseline: 105.9680x; 1.0065x over previous
import functools

import jax
import jax.numpy as jnp
from jax import lax
from jax.experimental import pallas as pl
from jax.experimental.pallas import tpu as pltpu
from jax.experimental.pallas import tpu_sc as plsc

N_NODES = 100000
N_EDGES = 6400000
N_GRAPHS = 128

ROW_W = 128
NW = 32
CH = 16
R_PAD = ((N_EDGES // ROW_W) + NW * CH - 1) // (NW * CH) * (NW * CH)
ROWS_PER_W = R_PAD // NW
NCH = ROWS_PER_W // CH
ACC = (N_GRAPHS + 1) * 16
TBL_R = ((N_NODES + 1) + 15) // 16 * 16


def _edge_var_partials(table, src2d, dst2d):
    mesh = plsc.VectorSubcoreMesh(core_axis_name="c", subcore_axis_name="s")

    @functools.partial(
        pl.kernel,
        mesh=mesh,
        compiler_params=pltpu.CompilerParams(
            needs_layout_passes=False, use_tc_tiling_on_sc=False
        ),
        out_type=(
            jax.ShapeDtypeStruct((NW, ACC), jnp.float32),
            jax.ShapeDtypeStruct((NW, ACC), jnp.float32),
        ),
        scratch_types=[
            pltpu.VMEM((2, CH, ROW_W), jnp.int32),
            pltpu.VMEM((2, CH, ROW_W), jnp.int32),
            pltpu.VMEM((2, CH * ROW_W, 8), jnp.float32),
            pltpu.VMEM((2, CH * ROW_W, 8), jnp.float32),
            pltpu.VMEM((ACC,), jnp.float32),
            pltpu.VMEM((ACC,), jnp.float32),
            pltpu.VMEM_SHARED((TBL_R, 8), jnp.float32),
            pltpu.SemaphoreType.DMA,
            pltpu.SemaphoreType.DMA,
        ],
    )
    def k(table_h, src_h, dst_h, sums_o, cnts_o,
          idx_s, idx_d, rows_s, rows_d, acc_s, acc_c, table_sp, sem_s, sem_d):
        cid = lax.axis_index("c")
        sid = lax.axis_index("s")
        wid = sid * 2 + cid

        part = TBL_R // 16
        pltpu.sync_copy(table_h.at[pl.ds(sid * part, part)],
                        table_sp.at[pl.ds(sid * part, part)])
        plsc.subcore_barrier()

        lanes = lax.iota(jnp.int32, 16)
        zeros16 = jnp.zeros((16,), jnp.float32)
        ones16 = jnp.ones((16,), jnp.float32)

        def zero_body(i, _):
            acc_s[pl.ds(i * 16, 16)] = zeros16
            acc_c[pl.ds(i * 16, 16)] = zeros16
            return 0

        lax.fori_loop(0, ACC // 16, zero_body, 0)

        start = wid * ROWS_PER_W

        def fire(t, buf):
            row = start + t * CH
            pltpu.sync_copy(src_h.at[pl.ds(row, CH)], idx_s.at[buf])
            pltpu.sync_copy(dst_h.at[pl.ds(row, CH)], idx_d.at[buf])
            for j in range(CH):
                pltpu.async_copy(table_sp.at[idx_s.at[buf, j]],
                                 rows_s.at[buf, pl.ds(j * ROW_W, ROW_W)], sem_s)
                pltpu.async_copy(table_sp.at[idx_d.at[buf, j]],
                                 rows_d.at[buf, pl.ds(j * ROW_W, ROW_W)], sem_d)

        def drain(buf):
            for j in range(CH):
                pltpu.make_async_copy(
                    table_sp.at[idx_s.at[buf, j]],
                    rows_s.at[buf, pl.ds(j * ROW_W, ROW_W)], sem_s).wait()
                pltpu.make_async_copy(
                    table_sp.at[idx_d.at[buf, j]],
                    rows_d.at[buf, pl.ds(j * ROW_W, ROW_W)], sem_d).wait()

        magic = jnp.full((16,), 0x5F3759DF, jnp.int32)
        one_i = jnp.full((16,), 1, jnp.int32)
        half = jnp.full((16,), 0.5, jnp.float32)
        threehalf = jnp.full((16,), 1.5, jnp.float32)
        col0 = jnp.full((16,), 0, jnp.int32)
        col1 = jnp.full((16,), 1, jnp.int32)
        col2 = jnp.full((16,), 2, jnp.int32)
        col3 = jnp.full((16,), 3, jnp.int32)

        fire(0, 0)

        def chunk_body(t, _):
            buf = lax.rem(t, 2)
            drain(buf)

            @pl.when(t + 1 < NCH)
            def _():
                fire(t + 1, 1 - buf)

            srows = rows_s.at[buf]
            drows = rows_d.at[buf]
            for j in range(CH * ROW_W // 16):
                ridx = lanes + (j * 16)
                sx = plsc.load_gather(srows, [ridx, col0])
                sy = plsc.load_gather(srows, [ridx, col1])
                sz = plsc.load_gather(srows, [ridx, col2])
                sb = plsc.load_gather(srows, [ridx, col3])
                dx = plsc.load_gather(drows, [ridx, col0])
                dy = plsc.load_gather(drows, [ridx, col1])
                dz = plsc.load_gather(drows, [ridx, col2])
                ex = dx - sx
                ey = dy - sy
                ez = dz - sz
                d2 = ex * ex + ey * ey + ez * ez
                yi = magic - lax.shift_right_arithmetic(
                    plsc.bitcast(d2, jnp.int32), one_i)
                y = plsc.bitcast(yi, jnp.float32)
                hx = half * d2
                y = y * (threehalf - hx * y * y)
                y = y * (threehalf - hx * y * y)
                y = y * (threehalf - hx * y * y)
                eu = d2 * y
                em1 = eu - ones16
                ev = em1 * em1
                b = sb.astype(jnp.int32)
                slot = b * 16 + lanes
                plsc.addupdate_scatter(acc_s, [slot], ev)
                plsc.addupdate_scatter(acc_c, [slot], ones16)
            return 0

        lax.fori_loop(0, NCH, chunk_body, 0)

        pltpu.sync_copy(acc_s, sums_o.at[wid])
        pltpu.sync_copy(acc_c, cnts_o.at[wid])

    return k(table, src2d, dst2d)


def kernel(node_pos, raw_edge_index, batch):
    batch_i = batch.astype(jnp.int32)
    edges = raw_edge_index.astype(jnp.int32)
    sentinel = jnp.array([[0.0, 0.0, 0.0, float(N_GRAPHS)]], jnp.float32)
    table = jnp.concatenate(
        [
            jnp.concatenate(
                [node_pos.astype(jnp.float32),
                 batch_i.astype(jnp.float32)[:, None]], axis=1
            ),
            sentinel,
        ],
        axis=0,
    )
    table = jnp.concatenate(
        [table, jnp.zeros((N_NODES + 1, 4), jnp.float32)], axis=1
    )
    table = jnp.concatenate(
        [table, jnp.zeros((TBL_R - (N_NODES + 1), 8), jnp.float32)], axis=0
    )
    n_pad = R_PAD * ROW_W - N_EDGES
    pad = jnp.full((n_pad,), N_NODES, jnp.int32)
    src2d = jnp.concatenate([edges[0], pad]).reshape(R_PAD, ROW_W)
    dst2d = jnp.concatenate([edges[1], pad]).reshape(R_PAD, ROW_W)
    sums_p, cnts_p = _edge_var_partials(table, src2d, dst2d)
    s = sums_p.sum(axis=0)[: N_GRAPHS * 16].reshape(N_GRAPHS, 16).sum(axis=1)
    c = cnts_p.sum(axis=0)[: N_GRAPHS * 16].reshape(N_GRAPHS, 16).sum(axis=1)
    graph_var = jnp.where(c > 0, s / jnp.maximum(c, 1.0), 0.0)
    return jnp.mean(graph_var)


# speedup vs baseline: 142.2205x; 1.3421x over previous
import functools

import jax
import jax.numpy as jnp
from jax import lax
from jax.experimental import pallas as pl
from jax.experimental.pallas import tpu as pltpu
from jax.experimental.pallas import tpu_sc as plsc

N_NODES = 100000
N_EDGES = 6400000
N_GRAPHS = 128

ROW_W = 128
NW = 32
CH = 16
N_ROWS = N_EDGES // ROW_W
NCHT = N_ROWS // CH
CB = NCHT // NW
CX = NCHT - CB * NW
ACC = N_GRAPHS * 16
TBL_R = (N_NODES + 15) // 16 * 16


def _edge_var_partials(table, src2d, dst2d):
    mesh = plsc.VectorSubcoreMesh(core_axis_name="c", subcore_axis_name="s")

    @functools.partial(
        pl.kernel,
        mesh=mesh,
        compiler_params=pltpu.CompilerParams(
            needs_layout_passes=False, use_tc_tiling_on_sc=False
        ),
        out_type=(
            jax.ShapeDtypeStruct((NW, ACC), jnp.float32),
            jax.ShapeDtypeStruct((NW, ACC), jnp.float32),
        ),
        scratch_types=[
            pltpu.VMEM((2, CH, ROW_W), jnp.int32),
            pltpu.VMEM((2, CH, ROW_W), jnp.int32),
            pltpu.VMEM((2, CH * ROW_W, 8), jnp.float32),
            pltpu.VMEM((2, CH * ROW_W, 8), jnp.float32),
            pltpu.VMEM((ACC,), jnp.float32),
            pltpu.VMEM((ACC,), jnp.float32),
            pltpu.VMEM_SHARED((TBL_R, 8), jnp.float32),
            pltpu.SemaphoreType.DMA,
            pltpu.SemaphoreType.DMA,
            pltpu.SemaphoreType.DMA,
        ],
    )
    def k(table_h, src_h, dst_h, sums_o, cnts_o,
          idx_s, idx_d, rows_s, rows_d, acc_s, acc_c, table_sp,
          sem_s, sem_d, sem_i):
        cid = lax.axis_index("c")
        sid = lax.axis_index("s")
        wid = sid * 2 + cid

        part = TBL_R // 16
        pltpu.sync_copy(table_h.at[pl.ds(sid * part, part)],
                        table_sp.at[pl.ds(sid * part, part)])
        plsc.subcore_barrier()

        lanes = lax.iota(jnp.int32, 16)
        zeros16 = jnp.zeros((16,), jnp.float32)
        ones16 = jnp.ones((16,), jnp.float32)

        def zero_body(i, _):
            acc_s[pl.ds(i * 16, 16)] = zeros16
            acc_c[pl.ds(i * 16, 16)] = zeros16
            return 0

        lax.fori_loop(0, ACC // 16, zero_body, 0)

        start = wid * CB + jnp.minimum(wid, CX)
        count = CB + jnp.where(wid < CX, 1, 0)

        def idx_fetch(t, slot):
            row = (start + t) * CH
            a = pltpu.async_copy(src_h.at[pl.ds(row, CH)], idx_s.at[slot], sem_i)
            b = pltpu.async_copy(dst_h.at[pl.ds(row, CH)], idx_d.at[slot], sem_i)
            return a, b

        def idx_wait(slot):
            pltpu.make_async_copy(src_h.at[pl.ds(0, CH)], idx_s.at[slot],
                                  sem_i).wait()
            pltpu.make_async_copy(dst_h.at[pl.ds(0, CH)], idx_d.at[slot],
                                  sem_i).wait()

        def fire(islot, buf):
            for j in range(CH):
                pltpu.async_copy(table_sp.at[idx_s.at[islot, j]],
                                 rows_s.at[buf, pl.ds(j * ROW_W, ROW_W)], sem_s)
                pltpu.async_copy(table_sp.at[idx_d.at[islot, j]],
                                 rows_d.at[buf, pl.ds(j * ROW_W, ROW_W)], sem_d)

        def drain(buf):
            pltpu.make_async_copy(table_h.at[pl.ds(0, CH * ROW_W)],
                                  rows_s.at[buf], sem_s).wait()
            pltpu.make_async_copy(table_h.at[pl.ds(0, CH * ROW_W)],
                                  rows_d.at[buf], sem_d).wait()

        magic = jnp.full((16,), 0x5F3759DF, jnp.int32)
        one_i = jnp.full((16,), 1, jnp.int32)
        half = jnp.full((16,), 0.5, jnp.float32)
        threehalf = jnp.full((16,), 1.5, jnp.float32)
        col0 = jnp.full((16,), 0, jnp.int32)
        col1 = jnp.full((16,), 1, jnp.int32)
        col2 = jnp.full((16,), 2, jnp.int32)
        col3 = jnp.full((16,), 3, jnp.int32)

        idx_fetch(0, 0)
        idx_wait(0)
        fire(0, 0)

        @pl.when(count > 1)
        def _():
            idx_fetch(1, 1)

        def chunk_body(t, _):
            buf = lax.rem(t, 2)
            drain(buf)

            @pl.when(t + 1 < count)
            def _():
                islot = lax.rem(t + 1, 2)
                idx_wait(islot)
                fire(islot, 1 - buf)

            @pl.when(t + 2 < count)
            def _():
                idx_fetch(t + 2, lax.rem(t + 2, 2))

            srows = rows_s.at[buf]
            drows = rows_d.at[buf]
            for j in range(CH * ROW_W // 16):
                ridx = lanes + (j * 16)
                sx = plsc.load_gather(srows, [ridx, col0])
                sy = plsc.load_gather(srows, [ridx, col1])
                sz = plsc.load_gather(srows, [ridx, col2])
                sb = plsc.load_gather(srows, [ridx, col3])
                dx = plsc.load_gather(drows, [ridx, col0])
                dy = plsc.load_gather(drows, [ridx, col1])
                dz = plsc.load_gather(drows, [ridx, col2])
                ex = dx - sx
                ey = dy - sy
                ez = dz - sz
                d2 = ex * ex + ey * ey + ez * ez
                yi = magic - lax.shift_right_arithmetic(
                    plsc.bitcast(d2, jnp.int32), one_i)
                y = plsc.bitcast(yi, jnp.float32)
                hx = half * d2
                y = y * (threehalf - hx * y * y)
                y = y * (threehalf - hx * y * y)
                y = y * (threehalf - hx * y * y)
                eu = d2 * y
                em1 = eu - ones16
                ev = em1 * em1
                b = sb.astype(jnp.int32)
                slot = b * 16 + lanes
                plsc.addupdate_scatter(acc_s, [slot], ev)
                plsc.addupdate_scatter(acc_c, [slot], ones16)
            return 0

        lax.fori_loop(0, count, chunk_body, 0)

        pltpu.sync_copy(acc_s, sums_o.at[wid])
        pltpu.sync_copy(acc_c, cnts_o.at[wid])

    return k(table, src2d, dst2d)


def kernel(node_pos, raw_edge_index, batch):
    batch_i = batch.astype(jnp.int32)
    edges = raw_edge_index.astype(jnp.int32)
    table = jnp.concatenate(
        [
            node_pos.astype(jnp.float32),
            batch_i.astype(jnp.float32)[:, None],
            jnp.zeros((N_NODES, 4), jnp.float32),
        ],
        axis=1,
    )
    if TBL_R > N_NODES:
        table = jnp.concatenate(
            [table, jnp.zeros((TBL_R - N_NODES, 8), jnp.float32)], axis=0
        )
    src2d = edges[0].reshape(N_ROWS, ROW_W)
    dst2d = edges[1].reshape(N_ROWS, ROW_W)
    sums_p, cnts_p = _edge_var_partials(table, src2d, dst2d)
    s = sums_p.sum(axis=0).reshape(N_GRAPHS, 16).sum(axis=1)
    c = cnts_p.sum(axis=0).reshape(N_GRAPHS, 16).sum(axis=1)
    graph_var = jnp.where(c > 0, s / jnp.maximum(c, 1.0), 0.0)
    return jnp.mean(graph_var)


# speedup vs baseline: 158.6831x; 1.1158x over previous
import functools

import jax
import jax.numpy as jnp
from jax import lax
from jax.experimental import pallas as pl
from jax.experimental.pallas import tpu as pltpu
from jax.experimental.pallas import tpu_sc as plsc

N_NODES = 100000
N_EDGES = 6400000
N_GRAPHS = 128

ROW_W = 128
NW = 32
CH = 16
N_ROWS = N_EDGES // ROW_W
NCHT = N_ROWS // CH
CB = NCHT // NW
CX = NCHT - CB * NW
ACC = N_GRAPHS * 16
TBL_R = (N_NODES + 15) // 16 * 16


def _edge_var_partials(table, src2d, dst2d):
    mesh = plsc.VectorSubcoreMesh(core_axis_name="c", subcore_axis_name="s")

    @functools.partial(
        pl.kernel,
        mesh=mesh,
        compiler_params=pltpu.CompilerParams(
            needs_layout_passes=False, use_tc_tiling_on_sc=False
        ),
        out_type=(
            jax.ShapeDtypeStruct((NW, ACC), jnp.float32),
            jax.ShapeDtypeStruct((NW, ACC), jnp.float32),
        ),
        scratch_types=[
            pltpu.VMEM((2, CH, ROW_W), jnp.int32),
            pltpu.VMEM((2, CH, ROW_W), jnp.int32),
            pltpu.VMEM((2, CH * ROW_W, 8), jnp.float32),
            pltpu.VMEM((2, CH * ROW_W, 8), jnp.float32),
            pltpu.VMEM((ACC,), jnp.float32),
            pltpu.VMEM((ACC,), jnp.float32),
            pltpu.VMEM_SHARED((TBL_R, 8), jnp.float32),
            pltpu.SemaphoreType.DMA,
            pltpu.SemaphoreType.DMA,
            pltpu.SemaphoreType.DMA,
        ],
    )
    def k(table_h, src_h, dst_h, sums_o, cnts_o,
          idx_s, idx_d, rows_s, rows_d, acc_s, acc_c, table_sp,
          sem_s, sem_d, sem_i):
        cid = lax.axis_index("c")
        sid = lax.axis_index("s")
        wid = sid * 2 + cid

        part = TBL_R // 16
        pltpu.sync_copy(table_h.at[pl.ds(sid * part, part)],
                        table_sp.at[pl.ds(sid * part, part)])
        plsc.subcore_barrier()

        lanes = lax.iota(jnp.int32, 16)
        zeros16 = jnp.zeros((16,), jnp.float32)
        ones16 = jnp.ones((16,), jnp.float32)

        def zero_body(i, _):
            acc_s[pl.ds(i * 16, 16)] = zeros16
            acc_c[pl.ds(i * 16, 16)] = zeros16
            return 0

        lax.fori_loop(0, ACC // 16, zero_body, 0)

        start = wid * CB + jnp.minimum(wid, CX)
        count = CB + jnp.where(wid < CX, 1, 0)

        def idx_fetch(t, slot):
            row = (start + t) * CH
            a = pltpu.async_copy(src_h.at[pl.ds(row, CH)], idx_s.at[slot], sem_i)
            b = pltpu.async_copy(dst_h.at[pl.ds(row, CH)], idx_d.at[slot], sem_i)
            return a, b

        def idx_wait(slot):
            pltpu.make_async_copy(src_h.at[pl.ds(0, CH)], idx_s.at[slot],
                                  sem_i).wait()
            pltpu.make_async_copy(dst_h.at[pl.ds(0, CH)], idx_d.at[slot],
                                  sem_i).wait()

        def fire(islot, buf):
            for j in range(CH):
                pltpu.async_copy(table_sp.at[idx_s.at[islot, j]],
                                 rows_s.at[buf, pl.ds(j * ROW_W, ROW_W)], sem_s)
                pltpu.async_copy(table_sp.at[idx_d.at[islot, j]],
                                 rows_d.at[buf, pl.ds(j * ROW_W, ROW_W)], sem_d)

        def drain(buf):
            pltpu.make_async_copy(table_h.at[pl.ds(0, CH * ROW_W)],
                                  rows_s.at[buf], sem_s).wait()
            pltpu.make_async_copy(table_h.at[pl.ds(0, CH * ROW_W)],
                                  rows_d.at[buf], sem_d).wait()

        magic = jnp.full((16,), 0x5F3759DF, jnp.int32)
        one_i = jnp.full((16,), 1, jnp.int32)
        half = jnp.full((16,), 0.5, jnp.float32)
        threehalf = jnp.full((16,), 1.5, jnp.float32)
        col0 = jnp.full((16,), 0, jnp.int32)
        col1 = jnp.full((16,), 1, jnp.int32)
        col2 = jnp.full((16,), 2, jnp.int32)
        col3 = jnp.full((16,), 3, jnp.int32)

        idx_fetch(0, 0)
        idx_wait(0)
        fire(0, 0)

        @pl.when(count > 1)
        def _():
            idx_fetch(1, 1)

        def chunk_body(t, _):
            buf = lax.rem(t, 2)
            drain(buf)

            @pl.when(t + 1 < count)
            def _():
                islot = lax.rem(t + 1, 2)
                idx_wait(islot)
                fire(islot, 1 - buf)

            @pl.when(t + 2 < count)
            def _():
                idx_fetch(t + 2, lax.rem(t + 2, 2))

            srows = rows_s.at[buf]
            drows = rows_d.at[buf]
            for j in range(CH * ROW_W // 16):
                ridx = lanes + (j * 16)
                sx = plsc.load_gather(srows, [ridx, col0])
                sy = plsc.load_gather(srows, [ridx, col1])
                sz = plsc.load_gather(srows, [ridx, col2])
                sb = plsc.load_gather(srows, [ridx, col3])
                dx = plsc.load_gather(drows, [ridx, col0])
                dy = plsc.load_gather(drows, [ridx, col1])
                dz = plsc.load_gather(drows, [ridx, col2])
                ex = dx - sx
                ey = dy - sy
                ez = dz - sz
                d2 = ex * ex + ey * ey + ez * ez
                yi = magic - lax.shift_right_arithmetic(
                    plsc.bitcast(d2, jnp.int32), one_i)
                y = plsc.bitcast(yi, jnp.float32)
                hx = half * d2
                y = y * (threehalf - hx * y * y)
                y = y * (threehalf - hx * y * y)
                eu = d2 * y
                em1 = eu - ones16
                ev = em1 * em1
                b = sb.astype(jnp.int32)
                slot = b * 16 + lanes
                plsc.addupdate_scatter(acc_s, [slot], ev)
                plsc.addupdate_scatter(acc_c, [slot], ones16)
            return 0

        lax.fori_loop(0, count, chunk_body, 0)

        pltpu.sync_copy(acc_s, sums_o.at[wid])
        pltpu.sync_copy(acc_c, cnts_o.at[wid])

    return k(table, src2d, dst2d)


def kernel(node_pos, raw_edge_index, batch):
    batch_i = batch.astype(jnp.int32)
    edges = raw_edge_index.astype(jnp.int32)
    table = jnp.concatenate(
        [
            node_pos.astype(jnp.float32),
            batch_i.astype(jnp.float32)[:, None],
            jnp.zeros((N_NODES, 4), jnp.float32),
        ],
        axis=1,
    )
    if TBL_R > N_NODES:
        table = jnp.concatenate(
            [table, jnp.zeros((TBL_R - N_NODES, 8), jnp.float32)], axis=0
        )
    src2d = edges[0].reshape(N_ROWS, ROW_W)
    dst2d = edges[1].reshape(N_ROWS, ROW_W)
    sums_p, cnts_p = _edge_var_partials(table, src2d, dst2d)
    s = sums_p.sum(axis=0).reshape(N_GRAPHS, 16).sum(axis=1)
    c = cnts_p.sum(axis=0).reshape(N_GRAPHS, 16).sum(axis=1)
    graph_var = jnp.where(c > 0, s / jnp.maximum(c, 1.0), 0.0)
    return jnp.mean(graph_var)


# speedup vs baseline: 161.3754x; 1.0170x over previous
import functools

import jax
import jax.numpy as jnp
from jax import lax
from jax.experimental import pallas as pl
from jax.experimental.pallas import tpu as pltpu
from jax.experimental.pallas import tpu_sc as plsc

N_NODES = 100000
N_EDGES = 6400000
N_GRAPHS = 128

ROW_W = 128
NW = 32
CH = 16
N_ROWS = N_EDGES // ROW_W
NCHT = N_ROWS // CH
CB = NCHT // NW
CX = NCHT - CB * NW
ACC = N_GRAPHS * 16
TBL_R = (N_NODES + 15) // 16 * 16


def _edge_var_partials(table, src2d, dst2d):
    mesh = plsc.VectorSubcoreMesh(core_axis_name="c", subcore_axis_name="s")

    @functools.partial(
        pl.kernel,
        mesh=mesh,
        compiler_params=pltpu.CompilerParams(
            needs_layout_passes=False, use_tc_tiling_on_sc=False
        ),
        out_type=(
            jax.ShapeDtypeStruct((NW, ACC), jnp.float32),
            jax.ShapeDtypeStruct((NW, ACC), jnp.float32),
        ),
        scratch_types=[
            pltpu.VMEM((2, CH * ROW_W), jnp.int32),
            pltpu.VMEM((2, CH * ROW_W), jnp.int32),
            pltpu.VMEM((2, CH * ROW_W, 8), jnp.float32),
            pltpu.VMEM((2, CH * ROW_W, 8), jnp.float32),
            pltpu.VMEM((ACC,), jnp.float32),
            pltpu.VMEM((ACC,), jnp.float32),
            pltpu.VMEM_SHARED((TBL_R, 8), jnp.float32),
            pltpu.SemaphoreType.DMA,
            pltpu.SemaphoreType.DMA,
            pltpu.SemaphoreType.DMA,
        ],
    )
    def k(table_h, src_h, dst_h, sums_o, cnts_o,
          idx_s, idx_d, rows_s, rows_d, acc_s, acc_c, table_sp,
          sem_s, sem_d, sem_i):
        cid = lax.axis_index("c")
        sid = lax.axis_index("s")
        wid = sid * 2 + cid

        part = TBL_R // 16
        pltpu.sync_copy(table_h.at[pl.ds(sid * part, part)],
                        table_sp.at[pl.ds(sid * part, part)])
        plsc.subcore_barrier()

        lanes = lax.iota(jnp.int32, 16)
        zeros16 = jnp.zeros((16,), jnp.float32)
        ones16 = jnp.ones((16,), jnp.float32)

        def zero_body(i, _):
            acc_s[pl.ds(i * 16, 16)] = zeros16
            acc_c[pl.ds(i * 16, 16)] = zeros16
            return 0

        lax.fori_loop(0, ACC // 16, zero_body, 0)

        start = wid * CB + jnp.minimum(wid, CX)
        count = CB + jnp.where(wid < CX, 1, 0)

        def idx_fetch(t, slot):
            base = (start + t) * CH * ROW_W
            a = pltpu.async_copy(src_h.at[pl.ds(base, CH * ROW_W)],
                                 idx_s.at[slot], sem_i)
            b = pltpu.async_copy(dst_h.at[pl.ds(base, CH * ROW_W)],
                                 idx_d.at[slot], sem_i)
            return a, b

        def idx_wait(slot):
            pltpu.make_async_copy(src_h.at[pl.ds(0, CH * ROW_W)],
                                  idx_s.at[slot], sem_i).wait()
            pltpu.make_async_copy(dst_h.at[pl.ds(0, CH * ROW_W)],
                                  idx_d.at[slot], sem_i).wait()

        def fire(islot, buf):
            pltpu.async_copy(table_sp.at[idx_s.at[islot]],
                             rows_s.at[buf], sem_s)
            pltpu.async_copy(table_sp.at[idx_d.at[islot]],
                             rows_d.at[buf], sem_d)

        def drain(buf):
            pltpu.make_async_copy(table_h.at[pl.ds(0, CH * ROW_W)],
                                  rows_s.at[buf], sem_s).wait()
            pltpu.make_async_copy(table_h.at[pl.ds(0, CH * ROW_W)],
                                  rows_d.at[buf], sem_d).wait()

        magic = jnp.full((16,), 0x5F3759DF, jnp.int32)
        one_i = jnp.full((16,), 1, jnp.int32)
        half = jnp.full((16,), 0.5, jnp.float32)
        threehalf = jnp.full((16,), 1.5, jnp.float32)
        col0 = jnp.full((16,), 0, jnp.int32)
        col1 = jnp.full((16,), 1, jnp.int32)
        col2 = jnp.full((16,), 2, jnp.int32)
        col3 = jnp.full((16,), 3, jnp.int32)

        idx_fetch(0, 0)
        idx_wait(0)
        fire(0, 0)

        @pl.when(count > 1)
        def _():
            idx_fetch(1, 1)

        def chunk_body(t, _):
            buf = lax.rem(t, 2)
            drain(buf)

            @pl.when(t + 1 < count)
            def _():
                islot = lax.rem(t + 1, 2)
                idx_wait(islot)
                fire(islot, 1 - buf)

            @pl.when(t + 2 < count)
            def _():
                idx_fetch(t + 2, lax.rem(t + 2, 2))

            srows = rows_s.at[buf]
            drows = rows_d.at[buf]
            for j in range(CH * ROW_W // 16):
                ridx = lanes + (j * 16)
                sx = plsc.load_gather(srows, [ridx, col0])
                sy = plsc.load_gather(srows, [ridx, col1])
                sz = plsc.load_gather(srows, [ridx, col2])
                sb = plsc.load_gather(srows, [ridx, col3])
                dx = plsc.load_gather(drows, [ridx, col0])
                dy = plsc.load_gather(drows, [ridx, col1])
                dz = plsc.load_gather(drows, [ridx, col2])
                ex = dx - sx
                ey = dy - sy
                ez = dz - sz
                d2 = ex * ex + ey * ey + ez * ez
                yi = magic - lax.shift_right_arithmetic(
                    plsc.bitcast(d2, jnp.int32), one_i)
                y = plsc.bitcast(yi, jnp.float32)
                hx = half * d2
                y = y * (threehalf - hx * y * y)
                y = y * (threehalf - hx * y * y)
                eu = d2 * y
                em1 = eu - ones16
                ev = em1 * em1
                b = sb.astype(jnp.int32)
                slot = b * 16 + lanes
                plsc.addupdate_scatter(acc_s, [slot], ev)
                plsc.addupdate_scatter(acc_c, [slot], ones16)
            return 0

        lax.fori_loop(0, count, chunk_body, 0)

        pltpu.sync_copy(acc_s, sums_o.at[wid])
        pltpu.sync_copy(acc_c, cnts_o.at[wid])

    return k(table, src2d, dst2d)


def kernel(node_pos, raw_edge_index, batch):
    batch_i = batch.astype(jnp.int32)
    edges = raw_edge_index.astype(jnp.int32)
    table = jnp.concatenate(
        [
            node_pos.astype(jnp.float32),
            batch_i.astype(jnp.float32)[:, None],
            jnp.zeros((N_NODES, 4), jnp.float32),
        ],
        axis=1,
    )
    if TBL_R > N_NODES:
        table = jnp.concatenate(
            [table, jnp.zeros((TBL_R - N_NODES, 8), jnp.float32)], axis=0
        )
    sums_p, cnts_p = _edge_var_partials(table, edges[0], edges[1])
    s = sums_p.sum(axis=0).reshape(N_GRAPHS, 16).sum(axis=1)
    c = cnts_p.sum(axis=0).reshape(N_GRAPHS, 16).sum(axis=1)
    graph_var = jnp.where(c > 0, s / jnp.maximum(c, 1.0), 0.0)
    return jnp.mean(graph_var)
